# Initial kernel scaffold; baseline (speedup 1.0000x reference)
import jax
import jax.numpy as jnp
from jax.experimental import pallas as pl


def kernel(x, expert_weights_up, expert_weights_down, expert_biases_up, expert_biases_down, router_weight):
    raise NotImplementedError("write your pallas kernel here")


# speedup vs baseline: 3.9899x; 3.9899x over previous
import jax
import jax.numpy as jnp
from jax.experimental import pallas as pl
from jax.experimental.pallas import tpu as pltpu

U_BLK = 1024


def _moe_body(x_ref, rw_ref, bu_ref, bd_ref, wup_ref, wdn_ref, out_ref,
              gate_ref):
    e = pl.program_id(0)
    u = pl.program_id(1)
    nu = pl.num_programs(1)
    n_e = pl.num_programs(0)

    @pl.when((e == 0) & (u == 0))
    def _init():
        x = x_ref[...]
        rw = rw_ref[...]
        logits = jnp.dot(x, rw.T, preferred_element_type=jnp.float32)
        col = jax.lax.broadcasted_iota(jnp.int32, logits.shape, 1)
        m1 = jnp.max(logits, axis=1, keepdims=True)
        i1 = jnp.min(jnp.where(logits == m1, col, n_e), axis=1, keepdims=True)
        masked = jnp.where(col == i1, -jnp.inf, logits)
        m2 = jnp.max(masked, axis=1, keepdims=True)
        i2 = jnp.min(jnp.where(masked == m2, col, n_e), axis=1, keepdims=True)
        t = jnp.exp(m2 - m1)
        w1 = 1.0 / (1.0 + t)
        w2 = t / (1.0 + t)
        gates = w1 * (col == i1) + w2 * (col == i2)
        gate_ref[...] = gates.T
        out_ref[...] = jnp.zeros_like(out_ref)

    x = x_ref[...]
    g = gate_ref[e, :].reshape(-1, 1)
    wup = wup_ref[0]
    h = jnp.dot(x, wup.T, preferred_element_type=jnp.float32)
    h = h + bu_ref[e, pl.ds(u * U_BLK, U_BLK)]
    h = 0.5 * h * (1.0 + jax.lax.erf(h * 0.7071067811865476))
    h = h * g
    wdn = wdn_ref[0]
    out_ref[...] += jnp.dot(h, wdn.T, preferred_element_type=jnp.float32)

    @pl.when(u == nu - 1)
    def _bias_down():
        out_ref[...] += g * bd_ref[e, :]


def kernel(x, expert_weights_up, expert_weights_down, expert_biases_up,
           expert_biases_down, router_weight):
    B, S, D = x.shape
    E, U, _ = expert_weights_up.shape
    x2d = x.reshape(B * S, D)
    nu = U // U_BLK

    out = pl.pallas_call(
        _moe_body,
        grid=(E, nu),
        in_specs=[
            pl.BlockSpec((B * S, D), lambda e, u: (0, 0)),
            pl.BlockSpec((E, D), lambda e, u: (0, 0)),
            pl.BlockSpec((E, U), lambda e, u: (0, 0)),
            pl.BlockSpec((E, D), lambda e, u: (0, 0)),
            pl.BlockSpec((1, U_BLK, D), lambda e, u: (e, u, 0)),
            pl.BlockSpec((1, D, U_BLK), lambda e, u: (e, 0, u)),
        ],
        out_specs=pl.BlockSpec((B * S, D), lambda e, u: (0, 0)),
        out_shape=jax.ShapeDtypeStruct((B * S, D), jnp.float32),
        scratch_shapes=[pltpu.VMEM((E, B * S), jnp.float32)],
    )(x2d, router_weight, expert_biases_up, expert_biases_down,
      expert_weights_up, expert_weights_down)
    return out.reshape(B, S, D)


# speedup vs baseline: 4.3650x; 1.0940x over previous
import jax
import jax.numpy as jnp
from jax.experimental import pallas as pl
from jax.experimental.pallas import tpu as pltpu

U_BLK = 1536


def _moe_body(x_ref, rw_ref, bu_ref, bd_ref, wup_ref, wdn_ref, out_ref,
              gate_ref):
    e = pl.program_id(0)
    u = pl.program_id(1)
    nu = pl.num_programs(1)
    n_e = pl.num_programs(0)

    @pl.when((e == 0) & (u == 0))
    def _init():
        x = x_ref[...]
        rw = rw_ref[...]
        logits = jnp.dot(x, rw.T, preferred_element_type=jnp.float32)
        col = jax.lax.broadcasted_iota(jnp.int32, logits.shape, 1)
        m1 = jnp.max(logits, axis=1, keepdims=True)
        i1 = jnp.min(jnp.where(logits == m1, col, n_e), axis=1, keepdims=True)
        masked = jnp.where(col == i1, -jnp.inf, logits)
        m2 = jnp.max(masked, axis=1, keepdims=True)
        i2 = jnp.min(jnp.where(masked == m2, col, n_e), axis=1, keepdims=True)
        t = jnp.exp(m2 - m1)
        w1 = 1.0 / (1.0 + t)
        w2 = t / (1.0 + t)
        gates = w1 * (col == i1) + w2 * (col == i2)
        gate_ref[...] = gates.T
        out_ref[...] = jnp.zeros_like(out_ref)

    x = x_ref[...]
    g = gate_ref[e, :].reshape(-1, 1)
    wup = wup_ref[0]
    h = jnp.dot(x, wup.T, preferred_element_type=jnp.float32)
    h = h + bu_ref[e, pl.ds(u * U_BLK, U_BLK)]
    h = 0.5 * h * (1.0 + jax.lax.erf(h * 0.7071067811865476))
    h = h * g
    wdn = wdn_ref[0]
    out_ref[...] += jnp.dot(h, wdn.T, preferred_element_type=jnp.float32)

    @pl.when(u == nu - 1)
    def _bias_down():
        out_ref[...] += g * bd_ref[e, :]


def kernel(x, expert_weights_up, expert_weights_down, expert_biases_up,
           expert_biases_down, router_weight):
    B, S, D = x.shape
    E, U, _ = expert_weights_up.shape
    x2d = x.reshape(B * S, D)
    nu = U // U_BLK

    out = pl.pallas_call(
        _moe_body,
        grid=(E, nu),
        in_specs=[
            pl.BlockSpec((B * S, D), lambda e, u: (0, 0)),
            pl.BlockSpec((E, D), lambda e, u: (0, 0)),
            pl.BlockSpec((E, U), lambda e, u: (0, 0)),
            pl.BlockSpec((E, D), lambda e, u: (0, 0)),
            pl.BlockSpec((1, U_BLK, D), lambda e, u: (e, u, 0)),
            pl.BlockSpec((1, D, U_BLK), lambda e, u: (e, 0, u)),
        ],
        out_specs=pl.BlockSpec((B * S, D), lambda e, u: (0, 0)),
        out_shape=jax.ShapeDtypeStruct((B * S, D), jnp.float32),
        scratch_shapes=[pltpu.VMEM((E, B * S), jnp.float32)],
    )(x2d, router_weight, expert_biases_up, expert_biases_down,
      expert_weights_up, expert_weights_down)
    return out.reshape(B, S, D)
